# baseline (speedup 1.0000x reference)
import functools

import jax
import jax.numpy as jnp
from jax import lax
from jax.experimental import pallas as pl
from jax.experimental.pallas import tpu as pltpu
from jax.experimental.pallas import tpu_sc as plsc

N = 10000
D = 128
E = 320000

NC = 2
NS = 16
NW = NC * NS
RPW = 320
NPAD = NW * RPW

CHUNK = 8000
NCHUNK = E // CHUNK
STAGE = CHUNK + 32
CAPP = E + STAGE + 16
SB = 2048
GSZ = 128
NSUB = SB // GSZ


@functools.cache
def _sc_kernels():
  mesh = plsc.VectorSubcoreMesh(core_axis_name="c", subcore_axis_name="s",
                                num_cores=NC, num_subcores=NS)

  @functools.partial(
      pl.kernel,
      out_type=(
          jax.ShapeDtypeStruct((NW * CAPP,), jnp.int32),
          jax.ShapeDtypeStruct((NW * CAPP,), jnp.int32),
          jax.ShapeDtypeStruct((NW * 16,), jnp.int32),
      ),
      mesh=mesh,
      compiler_params=pltpu.CompilerParams(needs_layout_passes=False),
      scratch_types=[
          pltpu.VMEM((CHUNK,), jnp.int32),
          pltpu.VMEM((CHUNK,), jnp.int32),
          pltpu.VMEM((STAGE,), jnp.int32),
          pltpu.VMEM((STAGE,), jnp.int32),
          pltpu.VMEM((16,), jnp.int32),
      ],
  )
  def _compact(src_hbm, dst_hbm, srcc_hbm, dstc_hbm, cnt_hbm,
               s_v, d_v, st_s, st_d, cnt_v):
    wid = lax.axis_index("s") * NC + lax.axis_index("c")
    lo = wid * RPW
    hi = lo + RPW
    lanes = lax.iota(jnp.int32, 16)

    zero16 = jnp.zeros((16,), jnp.int32)

    def zinit(i, _):
      st_s[pl.ds(i * 16, 16)] = zero16
      st_d[pl.ds(i * 16, 16)] = zero16
      return 0

    lax.fori_loop(0, STAGE // 16, zinit, 0)

    def shift_up(v, sh):
      idxs = jnp.maximum(lanes - sh, 0)
      g = lax.gather(v, idxs[:, None],
                     lax.GatherDimensionNumbers((), (0,), (0,)), (1,),
                     mode=lax.GatherScatterMode.PROMISE_IN_BOUNDS)
      return jnp.where(lanes >= sh, g, 0)

    def chunk_body(ci, carry):
      total, rem = carry
      pltpu.sync_copy(src_hbm.at[pl.ds(pl.multiple_of(ci * CHUNK, 8), CHUNK)], s_v)
      pltpu.sync_copy(dst_hbm.at[pl.ds(pl.multiple_of(ci * CHUNK, 8), CHUNK)], d_v)

      def grp(g, off):
        dd = d_v[pl.ds(g * 16, 16)]
        ss = s_v[pl.ds(g * 16, 16)]
        m = (dd >= lo) & (dd < hi)
        mi = m.astype(jnp.int32)
        p = mi
        for sh in (1, 2, 4, 8):
          p = p + shift_up(p, sh)
        pos = off + p - mi
        plsc.store_scatter(st_s, [pos], ss, mask=m)
        plsc.store_scatter(st_d, [pos], dd - lo, mask=m)
        return off + plsc.all_reduce_population_count(m)

      offv = lax.fori_loop(0, CHUNK // 16, grp,
                           jnp.full((16,), rem, jnp.int32))
      off = offv[0]
      fo = pl.multiple_of(wid * CAPP + total, 8)
      pltpu.sync_copy(st_s, srcc_hbm.at[pl.ds(fo, STAGE)])
      pltpu.sync_copy(st_d, dstc_hbm.at[pl.ds(fo, STAGE)])
      f = (off // 16) * 16
      ts = st_s[pl.ds(f, 16)]
      td = st_d[pl.ds(f, 16)]
      st_s[pl.ds(0, 16)] = ts
      st_d[pl.ds(0, 16)] = td
      return (total + f, off - f)

    total, rem = lax.fori_loop(0, NCHUNK, chunk_body,
                               (jnp.int32(0), jnp.int32(0)))
    padv = jnp.full((16,), RPW, jnp.int32)

    def pad(k, _):
      plsc.store_scatter(st_d, [rem + k * 16 + lanes], padv)
      return 0

    lax.fori_loop(0, SB // 16, pad, 0)
    fo = pl.multiple_of(wid * CAPP + total, 8)
    pltpu.sync_copy(st_s, srcc_hbm.at[pl.ds(fo, STAGE)])
    pltpu.sync_copy(st_d, dstc_hbm.at[pl.ds(fo, STAGE)])
    cnt_v[pl.ds(0, 16)] = jnp.full((16,), total + rem, jnp.int32)
    pltpu.sync_copy(cnt_v, cnt_hbm.at[pl.ds(pl.multiple_of(wid * 16, 8), 16)])

  @functools.partial(
      pl.kernel,
      out_type=jax.ShapeDtypeStruct((NPAD, D), jnp.float32),
      mesh=mesh,
      compiler_params=pltpu.CompilerParams(needs_layout_passes=False),
      scratch_types=[
          pltpu.VMEM((RPW + 8, D), jnp.float32),
          pltpu.VMEM((GSZ,), jnp.int32),
          pltpu.VMEM((GSZ,), jnp.int32),
          pltpu.VMEM((SB,), jnp.int32),
          pltpu.VMEM((GSZ, D), jnp.float32),
          pltpu.VMEM((GSZ, D), jnp.float32),
          pltpu.VMEM((16,), jnp.int32),
          pltpu.SemaphoreType.DMA,
          pltpu.SemaphoreType.DMA,
      ],
  )
  def _segmax(m_hbm, srcc_hbm, dstc_hbm, cnt_hbm, agg_hbm,
              acc, ib0, ib1, db, rows0, rows1, cnt_v, sem0, sem1):
    wid = lax.axis_index("s") * NC + lax.axis_index("c")

    pltpu.sync_copy(cnt_hbm.at[pl.ds(pl.multiple_of(wid * 16, 8), 16)], cnt_v)
    cnt = cnt_v[pl.ds(0, 16)][0]

    zrow = jnp.zeros((16,), jnp.float32)

    def z(r, _):
      for c in range(D // 16):
        acc[r, pl.ds(c * 16, 16)] = zrow
      return 0

    lax.fori_loop(0, RPW, z, 0)

    def fetch_idx(s, kk, ib):
      o = pl.multiple_of(wid * CAPP + s * SB + kk * GSZ, 8)
      pltpu.sync_copy(srcc_hbm.at[pl.ds(o, GSZ)], ib)

    def start_gather(ib, rw, sem):
      cp = pltpu.make_async_copy(m_hbm.at[ib], rw, sem)
      cp.start()

    def wait_gather(ib, rw, sem):
      pltpu.make_async_copy(m_hbm.at[ib], rw, sem).wait()

    def compute(kk, rw):
      def grp(k, _):
        dvec = db[pl.ds(kk * GSZ + k * 16, 16)]
        for j in range(16):
          d = dvec[j]
          a = [acc[d, pl.ds(c * 16, 16)] for c in range(D // 16)]
          v = [rw[k * 16 + j, pl.ds(c * 16, 16)] for c in range(D // 16)]
          for c in range(D // 16):
            acc[d, pl.ds(c * 16, 16)] = jnp.maximum(a[c], v[c])
        return 0

      lax.fori_loop(0, GSZ // 16, grp, 0)

    ns = (cnt + SB - 1) // SB

    def super_body(s, _):
      sb = pl.multiple_of(wid * CAPP + s * SB, 8)
      pltpu.sync_copy(dstc_hbm.at[pl.ds(sb, SB)], db)
      fetch_idx(s, 0, ib0)
      start_gather(ib0, rows0, sem0)

      def pair(p2, _):
        kk = p2 * 2
        fetch_idx(s, kk + 1, ib1)
        start_gather(ib1, rows1, sem1)
        wait_gather(ib0, rows0, sem0)
        compute(kk, rows0)

        @pl.when(p2 + 1 < NSUB // 2)
        def _():
          fetch_idx(s, kk + 2, ib0)
          start_gather(ib0, rows0, sem0)

        wait_gather(ib1, rows1, sem1)
        compute(kk + 1, rows1)
        return 0

      lax.fori_loop(0, NSUB // 2, pair, 0)
      return 0

    lax.fori_loop(0, ns, super_body, 0)
    pltpu.sync_copy(acc.at[pl.ds(0, RPW)],
                    agg_hbm.at[pl.ds(pl.multiple_of(wid * RPW, 8), RPW)])

  return _compact, _segmax


def _bn(t, g, b):
    mu = jnp.mean(t, axis=0)
    var = jnp.mean(t * t, axis=0) - mu * mu
    return g * (t - mu) * lax.rsqrt(var + 1e-5) + b


def _dense_a_body(x_ref, wp_ref, bp_ref, ws_ref, m_ref, s_ref):
    x = x_ref[...]
    m_ref[...] = jax.nn.relu(
        jnp.dot(x, wp_ref[...], preferred_element_type=jnp.float32)
        + bp_ref[...])
    s_ref[...] = jnp.dot(x, ws_ref[...], preferred_element_type=jnp.float32)


_dense_a = pl.pallas_call(
    _dense_a_body,
    out_shape=(jax.ShapeDtypeStruct((N, D), jnp.float32),
               jax.ShapeDtypeStruct((N, D), jnp.float32)),
)


def _dense_b_body(s_ref, agg_ref, wn_ref, b_ref, g_ref, be_ref,
                  wp2_ref, bp2_ref, ws2_ref, m2_ref, s2_ref):
    agg = agg_ref[pl.ds(0, N), :]
    t = (s_ref[...]
         + jnp.dot(agg, wn_ref[...], preferred_element_type=jnp.float32)
         + b_ref[...])
    h = jax.nn.relu(_bn(t, g_ref[...], be_ref[...]))
    m2_ref[...] = jax.nn.relu(
        jnp.dot(h, wp2_ref[...], preferred_element_type=jnp.float32)
        + bp2_ref[...])
    s2_ref[...] = jnp.dot(h, ws2_ref[...], preferred_element_type=jnp.float32)


_dense_b = pl.pallas_call(
    _dense_b_body,
    out_shape=(jax.ShapeDtypeStruct((N, D), jnp.float32),
               jax.ShapeDtypeStruct((N, D), jnp.float32)),
)


def _dense_c_body(s2_ref, agg2_ref, wn2_ref, b2_ref, g2_ref, be2_ref,
                  wh_ref, bh_ref, gh_ref, beh_ref,
                  wl_ref, bl_ref, gl_ref, bel_ref, out_ref):
    agg = agg2_ref[pl.ds(0, N), :]
    t = (s2_ref[...]
         + jnp.dot(agg, wn2_ref[...], preferred_element_type=jnp.float32)
         + b2_ref[...])
    h = jax.nn.relu(_bn(t, g2_ref[...], be2_ref[...]))
    h = jax.nn.relu(_bn(
        jnp.dot(h, wh_ref[...], preferred_element_type=jnp.float32)
        + bh_ref[...], gh_ref[...], beh_ref[...]))
    out_ref[...] = _bn(
        jnp.dot(h, wl_ref[...], preferred_element_type=jnp.float32)
        + bl_ref[...], gl_ref[...], bel_ref[...])


_dense_c = pl.pallas_call(
    _dense_c_body,
    out_shape=jax.ShapeDtypeStruct((N, D), jnp.float32),
)


def kernel(x, edge_index, Wp1, Ws1, Wn1, Wp2, Ws2, Wn2, Wh, Wl,
           bp1, b1, bp2, b2, bh, bl, be1, be2, beh, bel, g1, g2, gh, gl):
    compact, segmax = _sc_kernels()
    ei = edge_index.astype(jnp.int32)
    src = ei[0]
    dst = ei[1]

    srcc, dstc, cnts = compact(src, dst)
    m1, s1 = _dense_a(x, Wp1, bp1, Ws1)
    agg1 = segmax(m1, srcc, dstc, cnts)
    m2, s2 = _dense_b(s1, agg1, Wn1, b1, g1, be1, Wp2, bp2, Ws2)
    agg2 = segmax(m2, srcc, dstc, cnts)
    return _dense_c(s2, agg2, Wn2, b2, g2, be2,
                    Wh, bh, gh, beh, Wl, bl, gl, bel)


# speedup vs baseline: 1.6230x; 1.6230x over previous
import functools

import jax
import jax.numpy as jnp
from jax import lax
from jax.experimental import pallas as pl
from jax.experimental.pallas import tpu as pltpu
from jax.experimental.pallas import tpu_sc as plsc

N = 10000
D = 128
E = 320000

NC = 2
NS = 16
NW = NC * NS
RPW = 320
NPAD = NW * RPW

CHUNK = 8000
NCHUNK = E // CHUNK
STAGE = CHUNK + 32
CAPP = E + STAGE + 16
SB = 2048
GSZ = 128
NSUB = SB // GSZ


@functools.cache
def _sc_kernels():
  mesh = plsc.VectorSubcoreMesh(core_axis_name="c", subcore_axis_name="s",
                                num_cores=NC, num_subcores=NS)

  @functools.partial(
      pl.kernel,
      out_type=(
          jax.ShapeDtypeStruct((NW * CAPP,), jnp.int32),
          jax.ShapeDtypeStruct((NW * CAPP,), jnp.int32),
          jax.ShapeDtypeStruct((NW * 16,), jnp.int32),
      ),
      mesh=mesh,
      compiler_params=pltpu.CompilerParams(needs_layout_passes=False),
      scratch_types=[
          pltpu.VMEM((CHUNK,), jnp.int32),
          pltpu.VMEM((CHUNK,), jnp.int32),
          pltpu.VMEM((STAGE,), jnp.int32),
          pltpu.VMEM((STAGE,), jnp.int32),
          pltpu.VMEM((16,), jnp.int32),
      ],
  )
  def _compact(src_hbm, dst_hbm, srcc_hbm, dstc_hbm, cnt_hbm,
               s_v, d_v, st_s, st_d, cnt_v):
    wid = lax.axis_index("s") * NC + lax.axis_index("c")
    lo = wid * RPW
    hi = lo + RPW

    zero16 = jnp.zeros((16,), jnp.int32)

    def zinit(i, _):
      st_s[pl.ds(i * 16, 16)] = zero16
      st_d[pl.ds(i * 16, 16)] = zero16
      return 0

    lax.fori_loop(0, STAGE // 16, zinit, 0)

    def chunk_body(ci, carry):
      total, rem = carry
      pltpu.sync_copy(src_hbm.at[pl.ds(pl.multiple_of(ci * CHUNK, 8), CHUNK)], s_v)
      pltpu.sync_copy(dst_hbm.at[pl.ds(pl.multiple_of(ci * CHUNK, 8), CHUNK)], d_v)

      def grp(g, off):
        dd = d_v[pl.ds(g * 16, 16)]
        ss = s_v[pl.ds(g * 16, 16)]
        m = (dd >= lo) & (dd < hi)
        mi = m.astype(jnp.int32)
        pos = off + plsc.cumsum(mi) - mi
        plsc.store_scatter(st_s, [pos], ss, mask=m)
        plsc.store_scatter(st_d, [pos], dd - lo, mask=m)
        return off + jnp.sum(mi)

      off = lax.fori_loop(0, CHUNK // 16, grp, rem)
      fo = pl.multiple_of(wid * CAPP + total, 8)
      pltpu.sync_copy(st_s, srcc_hbm.at[pl.ds(fo, STAGE)])
      pltpu.sync_copy(st_d, dstc_hbm.at[pl.ds(fo, STAGE)])
      f = (off // 16) * 16
      ts = st_s[pl.ds(f, 16)]
      td = st_d[pl.ds(f, 16)]
      st_s[pl.ds(0, 16)] = ts
      st_d[pl.ds(0, 16)] = td
      return (total + f, off - f)

    total, rem = lax.fori_loop(0, NCHUNK, chunk_body,
                               (jnp.int32(0), jnp.int32(0)))
    fo = pl.multiple_of(wid * CAPP + total, 8)
    pltpu.sync_copy(st_s, srcc_hbm.at[pl.ds(fo, STAGE)])
    pltpu.sync_copy(st_d, dstc_hbm.at[pl.ds(fo, STAGE)])
    cnt_v[pl.ds(0, 16)] = jnp.full((16,), total + rem, jnp.int32)
    pltpu.sync_copy(cnt_v, cnt_hbm.at[pl.ds(pl.multiple_of(wid * 16, 8), 16)])

  G = GSZ

  @functools.partial(
      pl.kernel,
      out_type=jax.ShapeDtypeStruct((NPAD, D), jnp.float32),
      mesh=mesh,
      compiler_params=pltpu.CompilerParams(needs_layout_passes=False),
      scratch_types=[
          pltpu.VMEM((RPW, D), jnp.float32),
          pltpu.VMEM((GSZ,), jnp.int32),
          pltpu.VMEM((GSZ,), jnp.int32),
          pltpu.VMEM((GSZ, D), jnp.float32),
          pltpu.VMEM((16,), jnp.int32),
          pltpu.SemaphoreType.DMA,
      ],
  )
  def _segmax(m_hbm, srcc_hbm, dstc_hbm, cnt_hbm, agg_hbm,
              acc, idx_v, dl_v, rows_v, cnt_v, sem):
    wid = lax.axis_index("s") * NC + lax.axis_index("c")
    lanes = lax.iota(jnp.int32, 16)

    pltpu.sync_copy(cnt_hbm.at[pl.ds(pl.multiple_of(wid * 16, 8), 16)], cnt_v)
    cnt = jnp.sum(jnp.where(lanes == 0, cnt_v[pl.ds(0, 16)], 0))

    zrow = jnp.zeros((16,), jnp.float32)

    def z(r, _):
      for c in range(D // 16):
        acc[r, pl.ds(c * 16, 16)] = zrow
      return 0

    lax.fori_loop(0, RPW, z, 0)

    def batch(b, _):
      base = b * G
      bo = pl.multiple_of(wid * CAPP + base, 8)
      pltpu.sync_copy(srcc_hbm.at[pl.ds(bo, G)], idx_v)
      pltpu.sync_copy(dstc_hbm.at[pl.ds(bo, G)], dl_v)
      cp = pltpu.make_async_copy(m_hbm.at[idx_v], rows_v, sem)
      cp.start()
      cp.wait()

      def grp(k, _):
        dvec = dl_v[pl.ds(k * 16, 16)]
        for j in range(16):
          @pl.when(base + k * 16 + j < cnt)
          def _do():
            dloc = jnp.sum(jnp.where(lanes == j, dvec, 0))
            for c in range(D // 16):
              a = acc[dloc, pl.ds(c * 16, 16)]
              v = rows_v[k * 16 + j, pl.ds(c * 16, 16)]
              acc[dloc, pl.ds(c * 16, 16)] = jnp.maximum(a, v)
        return 0

      lax.fori_loop(0, G // 16, grp, 0)
      return 0

    nb = (cnt + G - 1) // G
    lax.fori_loop(0, nb, batch, 0)
    pltpu.sync_copy(acc, agg_hbm.at[pl.ds(pl.multiple_of(wid * RPW, 8), RPW)])

  return _compact, _segmax


def _bn(t, g, b):
    mu = jnp.mean(t, axis=0)
    var = jnp.mean(t * t, axis=0) - mu * mu
    return g * (t - mu) * lax.rsqrt(var + 1e-5) + b


def _dense_a_body(x_ref, wp_ref, bp_ref, ws_ref, m_ref, s_ref):
    x = x_ref[...]
    m_ref[...] = jax.nn.relu(
        jnp.dot(x, wp_ref[...], preferred_element_type=jnp.float32)
        + bp_ref[...])
    s_ref[...] = jnp.dot(x, ws_ref[...], preferred_element_type=jnp.float32)


_dense_a = pl.pallas_call(
    _dense_a_body,
    out_shape=(jax.ShapeDtypeStruct((N, D), jnp.float32),
               jax.ShapeDtypeStruct((N, D), jnp.float32)),
)


def _dense_b_body(s_ref, agg_ref, wn_ref, b_ref, g_ref, be_ref,
                  wp2_ref, bp2_ref, ws2_ref, m2_ref, s2_ref):
    agg = agg_ref[pl.ds(0, N), :]
    t = (s_ref[...]
         + jnp.dot(agg, wn_ref[...], preferred_element_type=jnp.float32)
         + b_ref[...])
    h = jax.nn.relu(_bn(t, g_ref[...], be_ref[...]))
    m2_ref[...] = jax.nn.relu(
        jnp.dot(h, wp2_ref[...], preferred_element_type=jnp.float32)
        + bp2_ref[...])
    s2_ref[...] = jnp.dot(h, ws2_ref[...], preferred_element_type=jnp.float32)


_dense_b = pl.pallas_call(
    _dense_b_body,
    out_shape=(jax.ShapeDtypeStruct((N, D), jnp.float32),
               jax.ShapeDtypeStruct((N, D), jnp.float32)),
)


def _dense_c_body(s2_ref, agg2_ref, wn2_ref, b2_ref, g2_ref, be2_ref,
                  wh_ref, bh_ref, gh_ref, beh_ref,
                  wl_ref, bl_ref, gl_ref, bel_ref, out_ref):
    agg = agg2_ref[pl.ds(0, N), :]
    t = (s2_ref[...]
         + jnp.dot(agg, wn2_ref[...], preferred_element_type=jnp.float32)
         + b2_ref[...])
    h = jax.nn.relu(_bn(t, g2_ref[...], be2_ref[...]))
    h = jax.nn.relu(_bn(
        jnp.dot(h, wh_ref[...], preferred_element_type=jnp.float32)
        + bh_ref[...], gh_ref[...], beh_ref[...]))
    out_ref[...] = _bn(
        jnp.dot(h, wl_ref[...], preferred_element_type=jnp.float32)
        + bl_ref[...], gl_ref[...], bel_ref[...])


_dense_c = pl.pallas_call(
    _dense_c_body,
    out_shape=jax.ShapeDtypeStruct((N, D), jnp.float32),
)


def kernel(x, edge_index, Wp1, Ws1, Wn1, Wp2, Ws2, Wn2, Wh, Wl,
           bp1, b1, bp2, b2, bh, bl, be1, be2, beh, bel, g1, g2, gh, gl):
    compact, segmax = _sc_kernels()
    ei = edge_index.astype(jnp.int32)
    src = ei[0]
    dst = ei[1]

    srcc, dstc, cnts = compact(src, dst)
    m1, s1 = _dense_a(x, Wp1, bp1, Ws1)
    agg1 = segmax(m1, srcc, dstc, cnts)
    m2, s2 = _dense_b(s1, agg1, Wn1, b1, g1, be1, Wp2, bp2, Ws2)
    agg2 = segmax(m2, srcc, dstc, cnts)
    return _dense_c(s2, agg2, Wn2, b2, g2, be2,
                    Wh, bh, gh, beh, Wl, bl, gl, bel)


# speedup vs baseline: 2.2570x; 1.3906x over previous
import functools

import jax
import jax.numpy as jnp
from jax import lax
from jax.experimental import pallas as pl
from jax.experimental.pallas import tpu as pltpu
from jax.experimental.pallas import tpu_sc as plsc

N = 10000
D = 128
E = 320000

NC = 2
NS = 16
NW = NC * NS
RPW = 320
NPAD = NW * RPW

CHUNK = 8000
NCHUNK = E // CHUNK
STAGE = CHUNK + 32
CAPP = E + STAGE + 16
SB = 2048
GSZ = 128
NSUB = SB // GSZ


@functools.cache
def _sc_kernels():
  mesh = plsc.VectorSubcoreMesh(core_axis_name="c", subcore_axis_name="s",
                                num_cores=NC, num_subcores=NS)

  @functools.partial(
      pl.kernel,
      out_type=(
          jax.ShapeDtypeStruct((NW * CAPP,), jnp.int32),
          jax.ShapeDtypeStruct((NW * CAPP,), jnp.int32),
          jax.ShapeDtypeStruct((NW * 16,), jnp.int32),
      ),
      mesh=mesh,
      compiler_params=pltpu.CompilerParams(needs_layout_passes=False),
      scratch_types=[
          pltpu.VMEM((CHUNK,), jnp.int32),
          pltpu.VMEM((CHUNK,), jnp.int32),
          pltpu.VMEM((STAGE,), jnp.int32),
          pltpu.VMEM((STAGE,), jnp.int32),
          pltpu.VMEM((16,), jnp.int32),
      ],
  )
  def _compact(src_hbm, dst_hbm, srcc_hbm, dstc_hbm, cnt_hbm,
               s_v, d_v, st_s, st_d, cnt_v):
    wid = lax.axis_index("s") * NC + lax.axis_index("c")
    lo = wid * RPW
    hi = lo + RPW

    zero16 = jnp.zeros((16,), jnp.int32)

    def zinit(i, _):
      st_s[pl.ds(i * 16, 16)] = zero16
      st_d[pl.ds(i * 16, 16)] = zero16
      return 0

    lax.fori_loop(0, STAGE // 16, zinit, 0)

    def chunk_body(ci, carry):
      total, rem = carry
      pltpu.sync_copy(src_hbm.at[pl.ds(pl.multiple_of(ci * CHUNK, 8), CHUNK)], s_v)
      pltpu.sync_copy(dst_hbm.at[pl.ds(pl.multiple_of(ci * CHUNK, 8), CHUNK)], d_v)

      def grp(g, off):
        dd = d_v[pl.ds(g * 16, 16)]
        ss = s_v[pl.ds(g * 16, 16)]
        m = (dd >= lo) & (dd < hi)
        mi = m.astype(jnp.int32)
        pos = off + plsc.cumsum(mi) - mi
        plsc.store_scatter(st_s, [pos], ss, mask=m)
        plsc.store_scatter(st_d, [pos], dd - lo, mask=m)
        return off + jnp.sum(mi)

      off = lax.fori_loop(0, CHUNK // 16, grp, rem)
      fo = pl.multiple_of(wid * CAPP + total, 8)
      pltpu.sync_copy(st_s, srcc_hbm.at[pl.ds(fo, STAGE)])
      pltpu.sync_copy(st_d, dstc_hbm.at[pl.ds(fo, STAGE)])
      f = (off // 16) * 16
      ts = st_s[pl.ds(f, 16)]
      td = st_d[pl.ds(f, 16)]
      st_s[pl.ds(0, 16)] = ts
      st_d[pl.ds(0, 16)] = td
      return (total + f, off - f)

    total, rem = lax.fori_loop(0, NCHUNK, chunk_body,
                               (jnp.int32(0), jnp.int32(0)))
    fo = pl.multiple_of(wid * CAPP + total, 8)
    pltpu.sync_copy(st_s, srcc_hbm.at[pl.ds(fo, STAGE)])
    pltpu.sync_copy(st_d, dstc_hbm.at[pl.ds(fo, STAGE)])
    cnt_v[pl.ds(0, 16)] = jnp.full((16,), total + rem, jnp.int32)
    pltpu.sync_copy(cnt_v, cnt_hbm.at[pl.ds(pl.multiple_of(wid * 16, 8), 16)])

  G = GSZ

  @functools.partial(
      pl.kernel,
      out_type=jax.ShapeDtypeStruct((NPAD, D), jnp.float32),
      mesh=mesh,
      compiler_params=pltpu.CompilerParams(needs_layout_passes=False),
      scratch_types=[
          pltpu.VMEM((RPW, D), jnp.float32),
          pltpu.VMEM((GSZ,), jnp.int32),
          pltpu.VMEM((GSZ,), jnp.int32),
          pltpu.VMEM((GSZ, D), jnp.float32),
          pltpu.VMEM((16,), jnp.int32),
          pltpu.SemaphoreType.DMA,
      ],
  )
  def _segmax(m_hbm, srcc_hbm, dstc_hbm, cnt_hbm, agg_hbm,
              acc, idx_v, dl_v, rows_v, cnt_v, sem):
    wid = lax.axis_index("s") * NC + lax.axis_index("c")
    lanes = lax.iota(jnp.int32, 16)

    pltpu.sync_copy(cnt_hbm.at[pl.ds(pl.multiple_of(wid * 16, 8), 16)], cnt_v)
    cnt = jnp.sum(jnp.where(lanes == 0, cnt_v[pl.ds(0, 16)], 0))

    zrow = jnp.zeros((16,), jnp.float32)

    def z(r, _):
      for c in range(D // 16):
        acc[r, pl.ds(c * 16, 16)] = zrow
      return 0

    lax.fori_loop(0, RPW, z, 0)

    def batch(b, _):
      base = b * G
      bo = pl.multiple_of(wid * CAPP + base, 8)
      pltpu.sync_copy(srcc_hbm.at[pl.ds(bo, G)], idx_v)
      pltpu.sync_copy(dstc_hbm.at[pl.ds(bo, G)], dl_v)
      cp = pltpu.make_async_copy(m_hbm.at[idx_v], rows_v, sem)
      cp.start()
      cp.wait()

      def grp(k, _):
        dvec = dl_v[pl.ds(k * 16, 16)]
        for j in range(16):
          @pl.when(base + k * 16 + j < cnt)
          def _do():
            dloc = dvec[j]
            a = [acc[dloc, pl.ds(c * 16, 16)] for c in range(D // 16)]
            v = [rows_v[k * 16 + j, pl.ds(c * 16, 16)] for c in range(D // 16)]
            for c in range(D // 16):
              acc[dloc, pl.ds(c * 16, 16)] = jnp.maximum(a[c], v[c])
        return 0

      lax.fori_loop(0, G // 16, grp, 0)
      return 0

    nb = (cnt + G - 1) // G
    lax.fori_loop(0, nb, batch, 0)
    pltpu.sync_copy(acc, agg_hbm.at[pl.ds(pl.multiple_of(wid * RPW, 8), RPW)])

  return _compact, _segmax


def _bn(t, g, b):
    mu = jnp.mean(t, axis=0)
    var = jnp.mean(t * t, axis=0) - mu * mu
    return g * (t - mu) * lax.rsqrt(var + 1e-5) + b


def _dense_a_body(x_ref, wp_ref, bp_ref, ws_ref, m_ref, s_ref):
    x = x_ref[...]
    m_ref[...] = jax.nn.relu(
        jnp.dot(x, wp_ref[...], preferred_element_type=jnp.float32)
        + bp_ref[...])
    s_ref[...] = jnp.dot(x, ws_ref[...], preferred_element_type=jnp.float32)


_dense_a = pl.pallas_call(
    _dense_a_body,
    out_shape=(jax.ShapeDtypeStruct((N, D), jnp.float32),
               jax.ShapeDtypeStruct((N, D), jnp.float32)),
)


def _dense_b_body(s_ref, agg_ref, wn_ref, b_ref, g_ref, be_ref,
                  wp2_ref, bp2_ref, ws2_ref, m2_ref, s2_ref):
    agg = agg_ref[pl.ds(0, N), :]
    t = (s_ref[...]
         + jnp.dot(agg, wn_ref[...], preferred_element_type=jnp.float32)
         + b_ref[...])
    h = jax.nn.relu(_bn(t, g_ref[...], be_ref[...]))
    m2_ref[...] = jax.nn.relu(
        jnp.dot(h, wp2_ref[...], preferred_element_type=jnp.float32)
        + bp2_ref[...])
    s2_ref[...] = jnp.dot(h, ws2_ref[...], preferred_element_type=jnp.float32)


_dense_b = pl.pallas_call(
    _dense_b_body,
    out_shape=(jax.ShapeDtypeStruct((N, D), jnp.float32),
               jax.ShapeDtypeStruct((N, D), jnp.float32)),
)


def _dense_c_body(s2_ref, agg2_ref, wn2_ref, b2_ref, g2_ref, be2_ref,
                  wh_ref, bh_ref, gh_ref, beh_ref,
                  wl_ref, bl_ref, gl_ref, bel_ref, out_ref):
    agg = agg2_ref[pl.ds(0, N), :]
    t = (s2_ref[...]
         + jnp.dot(agg, wn2_ref[...], preferred_element_type=jnp.float32)
         + b2_ref[...])
    h = jax.nn.relu(_bn(t, g2_ref[...], be2_ref[...]))
    h = jax.nn.relu(_bn(
        jnp.dot(h, wh_ref[...], preferred_element_type=jnp.float32)
        + bh_ref[...], gh_ref[...], beh_ref[...]))
    out_ref[...] = _bn(
        jnp.dot(h, wl_ref[...], preferred_element_type=jnp.float32)
        + bl_ref[...], gl_ref[...], bel_ref[...])


_dense_c = pl.pallas_call(
    _dense_c_body,
    out_shape=jax.ShapeDtypeStruct((N, D), jnp.float32),
)


def kernel(x, edge_index, Wp1, Ws1, Wn1, Wp2, Ws2, Wn2, Wh, Wl,
           bp1, b1, bp2, b2, bh, bl, be1, be2, beh, bel, g1, g2, gh, gl):
    compact, segmax = _sc_kernels()
    ei = edge_index.astype(jnp.int32)
    src = ei[0]
    dst = ei[1]

    srcc, dstc, cnts = compact(src, dst)
    m1, s1 = _dense_a(x, Wp1, bp1, Ws1)
    agg1 = segmax(m1, srcc, dstc, cnts)
    m2, s2 = _dense_b(s1, agg1, Wn1, b1, g1, be1, Wp2, bp2, Ws2)
    agg2 = segmax(m2, srcc, dstc, cnts)
    return _dense_c(s2, agg2, Wn2, b2, g2, be2,
                    Wh, bh, gh, beh, Wl, bl, gl, bel)


# speedup vs baseline: 2.2788x; 1.0097x over previous
import functools

import jax
import jax.numpy as jnp
from jax import lax
from jax.experimental import pallas as pl
from jax.experimental.pallas import tpu as pltpu
from jax.experimental.pallas import tpu_sc as plsc

N = 10000
D = 128
E = 320000

NC = 2
NS = 16
NW = NC * NS
RPW = 320
NPAD = NW * RPW

CHUNK = 8000
NCHUNK = E // CHUNK
STAGE = CHUNK + 32
CAPP = E + STAGE + 16
SB = 2048
GSZ = 128
NSUB = SB // GSZ


@functools.cache
def _sc_kernels():
  mesh = plsc.VectorSubcoreMesh(core_axis_name="c", subcore_axis_name="s",
                                num_cores=NC, num_subcores=NS)

  @functools.partial(
      pl.kernel,
      out_type=(
          jax.ShapeDtypeStruct((NW * CAPP,), jnp.int32),
          jax.ShapeDtypeStruct((NW * CAPP,), jnp.int32),
          jax.ShapeDtypeStruct((NW * 16,), jnp.int32),
      ),
      mesh=mesh,
      compiler_params=pltpu.CompilerParams(needs_layout_passes=False),
      scratch_types=[
          pltpu.VMEM((CHUNK,), jnp.int32),
          pltpu.VMEM((CHUNK,), jnp.int32),
          pltpu.VMEM((STAGE,), jnp.int32),
          pltpu.VMEM((STAGE,), jnp.int32),
          pltpu.VMEM((16,), jnp.int32),
      ],
  )
  def _compact(src_hbm, dst_hbm, srcc_hbm, dstc_hbm, cnt_hbm,
               s_v, d_v, st_s, st_d, cnt_v):
    wid = lax.axis_index("s") * NC + lax.axis_index("c")
    lo = wid * RPW
    hi = lo + RPW

    zero16 = jnp.zeros((16,), jnp.int32)

    def zinit(i, _):
      st_s[pl.ds(i * 16, 16)] = zero16
      st_d[pl.ds(i * 16, 16)] = zero16
      return 0

    lax.fori_loop(0, STAGE // 16, zinit, 0)

    def chunk_body(ci, carry):
      total, rem = carry
      pltpu.sync_copy(src_hbm.at[pl.ds(pl.multiple_of(ci * CHUNK, 8), CHUNK)], s_v)
      pltpu.sync_copy(dst_hbm.at[pl.ds(pl.multiple_of(ci * CHUNK, 8), CHUNK)], d_v)

      lane15 = jnp.full((16, 1), 15, jnp.int32)

      def grp(g, offv):
        dd = d_v[pl.ds(g * 16, 16)]
        ss = s_v[pl.ds(g * 16, 16)]
        m = (dd >= lo) & (dd < hi)
        mi = m.astype(jnp.int32)
        cs = plsc.cumsum(mi)
        pos = offv + cs - mi
        plsc.store_scatter(st_s, [pos], ss, mask=m)
        plsc.store_scatter(st_d, [pos], dd - lo, mask=m)
        tot = lax.gather(cs, lane15,
                         lax.GatherDimensionNumbers((), (0,), (0,)), (1,),
                         mode=lax.GatherScatterMode.PROMISE_IN_BOUNDS)
        return offv + tot

      offv = lax.fori_loop(0, CHUNK // 16, grp,
                           jnp.full((16,), rem, jnp.int32))
      off = offv[0]
      fo = pl.multiple_of(wid * CAPP + total, 8)
      pltpu.sync_copy(st_s, srcc_hbm.at[pl.ds(fo, STAGE)])
      pltpu.sync_copy(st_d, dstc_hbm.at[pl.ds(fo, STAGE)])
      f = (off // 16) * 16
      ts = st_s[pl.ds(f, 16)]
      td = st_d[pl.ds(f, 16)]
      st_s[pl.ds(0, 16)] = ts
      st_d[pl.ds(0, 16)] = td
      return (total + f, off - f)

    total, rem = lax.fori_loop(0, NCHUNK, chunk_body,
                               (jnp.int32(0), jnp.int32(0)))
    fo = pl.multiple_of(wid * CAPP + total, 8)
    pltpu.sync_copy(st_s, srcc_hbm.at[pl.ds(fo, STAGE)])
    pltpu.sync_copy(st_d, dstc_hbm.at[pl.ds(fo, STAGE)])
    cnt_v[pl.ds(0, 16)] = jnp.full((16,), total + rem, jnp.int32)
    pltpu.sync_copy(cnt_v, cnt_hbm.at[pl.ds(pl.multiple_of(wid * 16, 8), 16)])

  G = GSZ

  @functools.partial(
      pl.kernel,
      out_type=jax.ShapeDtypeStruct((NPAD, D), jnp.float32),
      mesh=mesh,
      compiler_params=pltpu.CompilerParams(needs_layout_passes=False),
      scratch_types=[
          pltpu.VMEM((RPW, D), jnp.float32),
          pltpu.VMEM((GSZ,), jnp.int32),
          pltpu.VMEM((GSZ,), jnp.int32),
          pltpu.VMEM((GSZ, D), jnp.float32),
          pltpu.VMEM((16,), jnp.int32),
          pltpu.SemaphoreType.DMA,
      ],
  )
  def _segmax(m_hbm, srcc_hbm, dstc_hbm, cnt_hbm, agg_hbm,
              acc, idx_v, dl_v, rows_v, cnt_v, sem):
    wid = lax.axis_index("s") * NC + lax.axis_index("c")
    lanes = lax.iota(jnp.int32, 16)

    pltpu.sync_copy(cnt_hbm.at[pl.ds(pl.multiple_of(wid * 16, 8), 16)], cnt_v)
    cnt = jnp.sum(jnp.where(lanes == 0, cnt_v[pl.ds(0, 16)], 0))

    zrow = jnp.zeros((16,), jnp.float32)

    def z(r, _):
      for c in range(D // 16):
        acc[r, pl.ds(c * 16, 16)] = zrow
      return 0

    lax.fori_loop(0, RPW, z, 0)

    def batch(b, _):
      base = b * G
      bo = pl.multiple_of(wid * CAPP + base, 8)
      pltpu.sync_copy(srcc_hbm.at[pl.ds(bo, G)], idx_v)
      pltpu.sync_copy(dstc_hbm.at[pl.ds(bo, G)], dl_v)
      cp = pltpu.make_async_copy(m_hbm.at[idx_v], rows_v, sem)
      cp.start()
      cp.wait()

      def grp(k, _):
        dvec = dl_v[pl.ds(k * 16, 16)]
        for j in range(16):
          @pl.when(base + k * 16 + j < cnt)
          def _do():
            dloc = dvec[j]
            a = [acc[dloc, pl.ds(c * 16, 16)] for c in range(D // 16)]
            v = [rows_v[k * 16 + j, pl.ds(c * 16, 16)] for c in range(D // 16)]
            for c in range(D // 16):
              acc[dloc, pl.ds(c * 16, 16)] = jnp.maximum(a[c], v[c])
        return 0

      lax.fori_loop(0, G // 16, grp, 0)
      return 0

    nb = (cnt + G - 1) // G
    lax.fori_loop(0, nb, batch, 0)
    pltpu.sync_copy(acc, agg_hbm.at[pl.ds(pl.multiple_of(wid * RPW, 8), RPW)])

  return _compact, _segmax


def _bn(t, g, b):
    mu = jnp.mean(t, axis=0)
    var = jnp.mean(t * t, axis=0) - mu * mu
    return g * (t - mu) * lax.rsqrt(var + 1e-5) + b


def _dense_a_body(x_ref, wp_ref, bp_ref, ws_ref, m_ref, s_ref):
    x = x_ref[...]
    m_ref[...] = jax.nn.relu(
        jnp.dot(x, wp_ref[...], preferred_element_type=jnp.float32)
        + bp_ref[...])
    s_ref[...] = jnp.dot(x, ws_ref[...], preferred_element_type=jnp.float32)


_dense_a = pl.pallas_call(
    _dense_a_body,
    out_shape=(jax.ShapeDtypeStruct((N, D), jnp.float32),
               jax.ShapeDtypeStruct((N, D), jnp.float32)),
)


def _dense_b_body(s_ref, agg_ref, wn_ref, b_ref, g_ref, be_ref,
                  wp2_ref, bp2_ref, ws2_ref, m2_ref, s2_ref):
    agg = agg_ref[pl.ds(0, N), :]
    t = (s_ref[...]
         + jnp.dot(agg, wn_ref[...], preferred_element_type=jnp.float32)
         + b_ref[...])
    h = jax.nn.relu(_bn(t, g_ref[...], be_ref[...]))
    m2_ref[...] = jax.nn.relu(
        jnp.dot(h, wp2_ref[...], preferred_element_type=jnp.float32)
        + bp2_ref[...])
    s2_ref[...] = jnp.dot(h, ws2_ref[...], preferred_element_type=jnp.float32)


_dense_b = pl.pallas_call(
    _dense_b_body,
    out_shape=(jax.ShapeDtypeStruct((N, D), jnp.float32),
               jax.ShapeDtypeStruct((N, D), jnp.float32)),
)


def _dense_c_body(s2_ref, agg2_ref, wn2_ref, b2_ref, g2_ref, be2_ref,
                  wh_ref, bh_ref, gh_ref, beh_ref,
                  wl_ref, bl_ref, gl_ref, bel_ref, out_ref):
    agg = agg2_ref[pl.ds(0, N), :]
    t = (s2_ref[...]
         + jnp.dot(agg, wn2_ref[...], preferred_element_type=jnp.float32)
         + b2_ref[...])
    h = jax.nn.relu(_bn(t, g2_ref[...], be2_ref[...]))
    h = jax.nn.relu(_bn(
        jnp.dot(h, wh_ref[...], preferred_element_type=jnp.float32)
        + bh_ref[...], gh_ref[...], beh_ref[...]))
    out_ref[...] = _bn(
        jnp.dot(h, wl_ref[...], preferred_element_type=jnp.float32)
        + bl_ref[...], gl_ref[...], bel_ref[...])


_dense_c = pl.pallas_call(
    _dense_c_body,
    out_shape=jax.ShapeDtypeStruct((N, D), jnp.float32),
)


def kernel(x, edge_index, Wp1, Ws1, Wn1, Wp2, Ws2, Wn2, Wh, Wl,
           bp1, b1, bp2, b2, bh, bl, be1, be2, beh, bel, g1, g2, gh, gl):
    compact, segmax = _sc_kernels()
    ei = edge_index.astype(jnp.int32)
    src = ei[0]
    dst = ei[1]

    srcc, dstc, cnts = compact(src, dst)
    m1, s1 = _dense_a(x, Wp1, bp1, Ws1)
    agg1 = segmax(m1, srcc, dstc, cnts)
    m2, s2 = _dense_b(s1, agg1, Wn1, b1, g1, be1, Wp2, bp2, Ws2)
    agg2 = segmax(m2, srcc, dstc, cnts)
    return _dense_c(s2, agg2, Wn2, b2, g2, be2,
                    Wh, bh, gh, beh, Wl, bl, gl, bel)


# speedup vs baseline: 2.7113x; 1.1898x over previous
import functools

import jax
import jax.numpy as jnp
from jax import lax
from jax.experimental import pallas as pl
from jax.experimental.pallas import tpu as pltpu
from jax.experimental.pallas import tpu_sc as plsc

N = 10000
D = 128
E = 320000

NC = 2
NS = 16
NW = NC * NS
RPW = 320
NPAD = NW * RPW

CHUNK = 8000
NCHUNK = E // CHUNK
STAGE = CHUNK + 32
CAPP = E + STAGE + 16
SB = 2048
GSZ = 128
NSUB = SB // GSZ


@functools.cache
def _sc_kernels():
  mesh = plsc.VectorSubcoreMesh(core_axis_name="c", subcore_axis_name="s",
                                num_cores=NC, num_subcores=NS)

  @functools.partial(
      pl.kernel,
      out_type=(
          jax.ShapeDtypeStruct((NW * CAPP,), jnp.int32),
          jax.ShapeDtypeStruct((NW * CAPP,), jnp.int32),
          jax.ShapeDtypeStruct((NW * 16,), jnp.int32),
      ),
      mesh=mesh,
      compiler_params=pltpu.CompilerParams(needs_layout_passes=False),
      scratch_types=[
          pltpu.VMEM((CHUNK,), jnp.int32),
          pltpu.VMEM((CHUNK,), jnp.int32),
          pltpu.VMEM((STAGE,), jnp.int32),
          pltpu.VMEM((STAGE,), jnp.int32),
          pltpu.VMEM((16,), jnp.int32),
      ],
  )
  def _compact(src_hbm, dst_hbm, srcc_hbm, dstc_hbm, cnt_hbm,
               s_v, d_v, st_s, st_d, cnt_v):
    wid = lax.axis_index("s") * NC + lax.axis_index("c")
    lo = wid * RPW
    hi = lo + RPW

    zero16 = jnp.zeros((16,), jnp.int32)

    def zinit(i, _):
      st_s[pl.ds(i * 16, 16)] = zero16
      st_d[pl.ds(i * 16, 16)] = zero16
      return 0

    lax.fori_loop(0, STAGE // 16, zinit, 0)

    def chunk_body(ci, carry):
      total, rem = carry
      pltpu.sync_copy(src_hbm.at[pl.ds(pl.multiple_of(ci * CHUNK, 8), CHUNK)], s_v)
      pltpu.sync_copy(dst_hbm.at[pl.ds(pl.multiple_of(ci * CHUNK, 8), CHUNK)], d_v)

      lane15 = jnp.full((16, 1), 15, jnp.int32)

      def grp(g, offv):
        dd = d_v[pl.ds(g * 16, 16)]
        ss = s_v[pl.ds(g * 16, 16)]
        m = (dd >= lo) & (dd < hi)
        mi = m.astype(jnp.int32)
        cs = plsc.cumsum(mi)
        pos = offv + cs - mi
        plsc.store_scatter(st_s, [pos], ss, mask=m)
        plsc.store_scatter(st_d, [pos], dd - lo, mask=m)
        tot = lax.gather(cs, lane15,
                         lax.GatherDimensionNumbers((), (0,), (0,)), (1,),
                         mode=lax.GatherScatterMode.PROMISE_IN_BOUNDS)
        return offv + tot

      offv = lax.fori_loop(0, CHUNK // 16, grp,
                           jnp.full((16,), rem, jnp.int32))
      off = offv[0]
      fo = pl.multiple_of(wid * CAPP + total, 8)
      pltpu.sync_copy(st_s, srcc_hbm.at[pl.ds(fo, STAGE)])
      pltpu.sync_copy(st_d, dstc_hbm.at[pl.ds(fo, STAGE)])
      f = (off // 16) * 16
      ts = st_s[pl.ds(f, 16)]
      td = st_d[pl.ds(f, 16)]
      st_s[pl.ds(0, 16)] = ts
      st_d[pl.ds(0, 16)] = td
      return (total + f, off - f)

    total, rem = lax.fori_loop(0, NCHUNK, chunk_body,
                               (jnp.int32(0), jnp.int32(0)))
    fo = pl.multiple_of(wid * CAPP + total, 8)
    pltpu.sync_copy(st_s, srcc_hbm.at[pl.ds(fo, STAGE)])
    pltpu.sync_copy(st_d, dstc_hbm.at[pl.ds(fo, STAGE)])
    cnt_v[pl.ds(0, 16)] = jnp.full((16,), total + rem, jnp.int32)
    pltpu.sync_copy(cnt_v, cnt_hbm.at[pl.ds(pl.multiple_of(wid * 16, 8), 16)])

  G = GSZ

  @functools.partial(
      pl.kernel,
      out_type=jax.ShapeDtypeStruct((NPAD, D), jnp.float32),
      mesh=mesh,
      compiler_params=pltpu.CompilerParams(needs_layout_passes=False),
      scratch_types=[
          pltpu.VMEM((RPW, D), jnp.float32),
          pltpu.VMEM((GSZ,), jnp.int32),
          pltpu.VMEM((GSZ,), jnp.int32),
          pltpu.VMEM((GSZ,), jnp.int32),
          pltpu.VMEM((GSZ,), jnp.int32),
          pltpu.VMEM((GSZ, D), jnp.float32),
          pltpu.VMEM((GSZ, D), jnp.float32),
          pltpu.VMEM((16,), jnp.int32),
          pltpu.SemaphoreType.DMA,
          pltpu.SemaphoreType.DMA,
      ],
  )
  def _segmax(m_hbm, srcc_hbm, dstc_hbm, cnt_hbm, agg_hbm,
              acc, idx_a, dl_a, idx_b, dl_b, rows_a, rows_b,
              cnt_v, sem0, sem1):
    wid = lax.axis_index("s") * NC + lax.axis_index("c")
    lanes = lax.iota(jnp.int32, 16)
    G = GSZ

    pltpu.sync_copy(cnt_hbm.at[pl.ds(pl.multiple_of(wid * 16, 8), 16)], cnt_v)
    cnt = jnp.sum(jnp.where(lanes == 0, cnt_v[pl.ds(0, 16)], 0))

    zrow = jnp.zeros((16,), jnp.float32)

    def z(r, _):
      for c in range(D // 16):
        acc[r, pl.ds(c * 16, 16)] = zrow
      return 0

    lax.fori_loop(0, RPW, z, 0)

    def fetch(b, idx_r, dl_r):
      bo = pl.multiple_of(wid * CAPP + b * G, 8)
      pltpu.sync_copy(srcc_hbm.at[pl.ds(bo, G)], idx_r)
      pltpu.sync_copy(dstc_hbm.at[pl.ds(bo, G)], dl_r)

    def compute(b, rows_r, dl_r):
      base = b * G

      def grp(k, _):
        dvec = dl_r[pl.ds(k * 16, 16)]
        for j in range(16):
          @pl.when(base + k * 16 + j < cnt)
          def _do():
            dloc = dvec[j]
            a = [acc[dloc, pl.ds(c * 16, 16)] for c in range(D // 16)]
            v = [rows_r[k * 16 + j, pl.ds(c * 16, 16)] for c in range(D // 16)]
            for c in range(D // 16):
              acc[dloc, pl.ds(c * 16, 16)] = jnp.maximum(a[c], v[c])
        return 0

      lax.fori_loop(0, G // 16, grp, 0)

    nb = (cnt + G - 1) // G
    npair = (nb + 1) // 2
    fetch(0, idx_a, dl_a)

    def pair(p2, _):
      b0 = p2 * 2
      b1 = b0 + 1
      cp0 = pltpu.make_async_copy(m_hbm.at[idx_a], rows_a, sem0)
      cp0.start()

      @pl.when(b1 < nb)
      def _():
        fetch(b1, idx_b, dl_b)

      cp1 = pltpu.make_async_copy(m_hbm.at[idx_b], rows_b, sem1)
      cp0.wait()

      @pl.when(b1 < nb)
      def _():
        cp1.start()

      compute(b0, rows_a, dl_a)

      @pl.when(p2 + 1 < npair)
      def _():
        fetch(b0 + 2, idx_a, dl_a)

      @pl.when(b1 < nb)
      def _():
        cp1.wait()
        compute(b1, rows_b, dl_b)

      return 0

    lax.fori_loop(0, npair, pair, 0)
    pltpu.sync_copy(acc, agg_hbm.at[pl.ds(pl.multiple_of(wid * RPW, 8), RPW)])

  return _compact, _segmax


def _bn(t, g, b):
    mu = jnp.mean(t, axis=0)
    var = jnp.mean(t * t, axis=0) - mu * mu
    return g * (t - mu) * lax.rsqrt(var + 1e-5) + b


def _dense_a_body(x_ref, wp_ref, bp_ref, ws_ref, m_ref, s_ref):
    x = x_ref[...]
    m_ref[...] = jax.nn.relu(
        jnp.dot(x, wp_ref[...], preferred_element_type=jnp.float32)
        + bp_ref[...])
    s_ref[...] = jnp.dot(x, ws_ref[...], preferred_element_type=jnp.float32)


_dense_a = pl.pallas_call(
    _dense_a_body,
    out_shape=(jax.ShapeDtypeStruct((N, D), jnp.float32),
               jax.ShapeDtypeStruct((N, D), jnp.float32)),
)


def _dense_b_body(s_ref, agg_ref, wn_ref, b_ref, g_ref, be_ref,
                  wp2_ref, bp2_ref, ws2_ref, m2_ref, s2_ref):
    agg = agg_ref[pl.ds(0, N), :]
    t = (s_ref[...]
         + jnp.dot(agg, wn_ref[...], preferred_element_type=jnp.float32)
         + b_ref[...])
    h = jax.nn.relu(_bn(t, g_ref[...], be_ref[...]))
    m2_ref[...] = jax.nn.relu(
        jnp.dot(h, wp2_ref[...], preferred_element_type=jnp.float32)
        + bp2_ref[...])
    s2_ref[...] = jnp.dot(h, ws2_ref[...], preferred_element_type=jnp.float32)


_dense_b = pl.pallas_call(
    _dense_b_body,
    out_shape=(jax.ShapeDtypeStruct((N, D), jnp.float32),
               jax.ShapeDtypeStruct((N, D), jnp.float32)),
)


def _dense_c_body(s2_ref, agg2_ref, wn2_ref, b2_ref, g2_ref, be2_ref,
                  wh_ref, bh_ref, gh_ref, beh_ref,
                  wl_ref, bl_ref, gl_ref, bel_ref, out_ref):
    agg = agg2_ref[pl.ds(0, N), :]
    t = (s2_ref[...]
         + jnp.dot(agg, wn2_ref[...], preferred_element_type=jnp.float32)
         + b2_ref[...])
    h = jax.nn.relu(_bn(t, g2_ref[...], be2_ref[...]))
    h = jax.nn.relu(_bn(
        jnp.dot(h, wh_ref[...], preferred_element_type=jnp.float32)
        + bh_ref[...], gh_ref[...], beh_ref[...]))
    out_ref[...] = _bn(
        jnp.dot(h, wl_ref[...], preferred_element_type=jnp.float32)
        + bl_ref[...], gl_ref[...], bel_ref[...])


_dense_c = pl.pallas_call(
    _dense_c_body,
    out_shape=jax.ShapeDtypeStruct((N, D), jnp.float32),
)


def kernel(x, edge_index, Wp1, Ws1, Wn1, Wp2, Ws2, Wn2, Wh, Wl,
           bp1, b1, bp2, b2, bh, bl, be1, be2, beh, bel, g1, g2, gh, gl):
    compact, segmax = _sc_kernels()
    ei = edge_index.astype(jnp.int32)
    src = ei[0]
    dst = ei[1]

    srcc, dstc, cnts = compact(src, dst)
    m1, s1 = _dense_a(x, Wp1, bp1, Ws1)
    agg1 = segmax(m1, srcc, dstc, cnts)
    m2, s2 = _dense_b(s1, agg1, Wn1, b1, g1, be1, Wp2, bp2, Ws2)
    agg2 = segmax(m2, srcc, dstc, cnts)
    return _dense_c(s2, agg2, Wn2, b2, g2, be2,
                    Wh, bh, gh, beh, Wl, bl, gl, bel)
